# baseline (speedup 1.0000x reference)
import jax
import jax.numpy as jnp
from jax.experimental import pallas as pl
from jax.experimental.pallas import tpu as pltpu
from jax.experimental.pallas import tpu_sc as plsc

LANES = 128
GW = 4
R = 256


def _sc_gather(n, v, table, idx_padded, chunk_table, chunk_idx):
    n_units = 32
    per_unit = n // n_units
    stride = 8

    @pl.kernel(
        out_type=[
            jax.ShapeDtypeStruct((n, v), jnp.float32),
            jax.ShapeDtypeStruct((n, 128), jnp.float32),
        ],
        mesh=plsc.VectorSubcoreMesh(core_axis_name="c", subcore_axis_name="s"),
        scratch_types=[
            pltpu.VMEM((per_unit // GW * stride,), jnp.int32),
            pltpu.VMEM((2, GW, v), jnp.float32),
            pltpu.VMEM((per_unit,), jnp.int32),
            pltpu.VMEM((per_unit, 128), jnp.float32),
            pltpu.SemaphoreType.DMA,
            pltpu.SemaphoreType.DMA,
            pltpu.SemaphoreType.DMA,
            pltpu.SemaphoreType.DMA,
            pltpu.SemaphoreType.DMA,
        ],
    )
    def gather_kernel(
        table_hbm,
        i_hbm,
        ct_hbm,
        ci_hbm,
        o_hbm,
        tch_hbm,
        idx_vmem,
        buf,
        cidx_vmem,
        cbuf,
        g0,
        g1,
        s0,
        s1,
        c0,
    ):
        core = jax.lax.axis_index("c")
        subcore = jax.lax.axis_index("s")
        u = core * 16 + subcore
        base = u * per_unit
        nwin = per_unit // GW

        pltpu.async_copy(
            i_hbm.at[0, pl.ds(u * (nwin * stride), nwin * stride)], idx_vmem, g0
        ).wait()
        pltpu.async_copy(ci_hbm.at[0, pl.ds(base, per_unit)], cidx_vmem, c0).wait()
        pltpu.make_async_copy(ct_hbm.at[cidx_vmem], cbuf, c0).start()

        def gather_dma(w, slot, sem):
            idx_win = idx_vmem.at[pl.ds(w * stride, GW)]
            return pltpu.make_async_copy(table_hbm.at[idx_win], buf.at[slot], sem)

        def write_dma(w, slot, sem):
            return pltpu.make_async_copy(
                buf.at[slot], o_hbm.at[pl.ds(base + w * GW, GW)], sem
            )

        gather_dma(0, 0, g0).start()

        @pl.loop(0, nwin, step=2)
        def _(w):
            gather_dma(w, 0, g0).wait()
            write_dma(w, 0, s0).start()

            @pl.when(w > 0)
            def _():
                write_dma(w - 1, 1, s1).wait()

            gather_dma(w + 1, 1, g1).start()
            gather_dma(w + 1, 1, g1).wait()
            write_dma(w + 1, 1, s1).start()

            @pl.when(w + 2 < nwin)
            def _():
                write_dma(w, 0, s0).wait()
                gather_dma(w + 2, 0, g0).start()

        write_dma(nwin - 2, 0, s0).wait()
        write_dma(nwin - 1, 1, s1).wait()
        pltpu.make_async_copy(ct_hbm.at[cidx_vmem], cbuf, c0).wait()
        pltpu.async_copy(cbuf, tch_hbm.at[pl.ds(base, per_unit)], c0).wait()

    return gather_kernel(table, idx_padded, chunk_table, chunk_idx)


def _ce_body(emb_ref, tch_ref, tlan_ref, loss_ref):
    step = pl.program_id(0)

    @pl.when(step == 0)
    def _():
        loss_ref[0, 0] = 0.0

    rows = emb_ref[...]
    r, v = rows.shape
    m = jnp.max(rows, axis=1, keepdims=True)
    s = jnp.sum(jnp.exp(rows - m), axis=1)
    lse = m.reshape(r) + jnp.log(s)
    tch = tch_ref[...]
    tlan = tlan_ref[...]
    li = jax.lax.broadcasted_iota(jnp.int32, (r, 128), 1)
    tval = jnp.sum(jnp.where(li == tlan, tch, 0.0), axis=1)
    loss_ref[0, 0] += jnp.sum(lse - tval)


def _tc_ce(n, v, emb, tch, tlancol):
    loss_sum = pl.pallas_call(
        _ce_body,
        grid=(n // R,),
        in_specs=[
            pl.BlockSpec((R, v), lambda i: (i, 0)),
            pl.BlockSpec((R, 128), lambda i: (i, 0)),
            pl.BlockSpec((R, 1), lambda i: (i, 0)),
        ],
        out_specs=pl.BlockSpec((1, 1), lambda i: (0, 0), memory_space=pltpu.SMEM),
        out_shape=jax.ShapeDtypeStruct((1, 1), jnp.float32),
    )(emb, tch, tlancol)
    return loss_sum[0, 0]


def kernel(indices, targets, table):
    b, t = indices.shape
    n = b * t
    vocab, v = table.shape
    sub = v // LANES
    idx_flat = indices.reshape(n)
    tgt_flat = targets.reshape(n)
    idx_win4 = indices.reshape(n // GW, GW)
    idx_padded = jnp.pad(idx_win4, ((0, 0), (0, 8 - GW))).reshape(1, n // GW * 8)
    chunk_table = table.reshape(vocab * v // 128, 128)
    chunk_idx = (idx_flat * (v // 128) + tgt_flat // 128).reshape(1, n)
    tlancol = (tgt_flat % 128).reshape(n, 1)

    emb, tch = _sc_gather(n, v, table, idx_padded, chunk_table, chunk_idx)
    loss_sum = _tc_ce(n, v, emb, tch, tlancol)
    embeddings = emb.reshape(b, t, v)
    loss = loss_sum / n
    return (embeddings, loss)


# speedup vs baseline: 2.6538x; 2.6538x over previous
import jax
import jax.numpy as jnp
from jax.experimental import pallas as pl
from jax.experimental.pallas import tpu as pltpu
from jax.experimental.pallas import tpu_sc as plsc

LANES = 128
GW = 4
R = 128


def _sc_gather(n, v, table, idx_padded):
    n_units = 32
    per_unit = n // n_units
    stride = 8

    @pl.kernel(
        out_type=jax.ShapeDtypeStruct((n, v), jnp.float32),
        mesh=plsc.VectorSubcoreMesh(core_axis_name="c", subcore_axis_name="s"),
        scratch_types=[
            pltpu.VMEM((per_unit // GW * stride,), jnp.int32),
            pltpu.VMEM((2, GW, v), jnp.float32),
            pltpu.SemaphoreType.DMA,
            pltpu.SemaphoreType.DMA,
            pltpu.SemaphoreType.DMA,
            pltpu.SemaphoreType.DMA,
        ],
    )
    def gather_kernel(table_hbm, i_hbm, o_hbm, idx_vmem, buf, g0, g1, s0, s1):
        core = jax.lax.axis_index("c")
        subcore = jax.lax.axis_index("s")
        u = core * 16 + subcore
        base = u * per_unit
        nwin = per_unit // GW

        pltpu.async_copy(
            i_hbm.at[0, pl.ds(u * (nwin * stride), nwin * stride)], idx_vmem, g0
        ).wait()

        def gather_dma(w, slot, sem):
            idx_win = idx_vmem.at[pl.ds(w * stride, GW)]
            return pltpu.make_async_copy(table_hbm.at[idx_win], buf.at[slot], sem)

        def write_dma(w, slot, sem):
            return pltpu.make_async_copy(
                buf.at[slot], o_hbm.at[pl.ds(base + w * GW, GW)], sem
            )

        gather_dma(0, 0, g0).start()

        @pl.loop(0, nwin, step=2)
        def _(w):
            gather_dma(w, 0, g0).wait()
            write_dma(w, 0, s0).start()

            @pl.when(w > 0)
            def _():
                write_dma(w - 1, 1, s1).wait()

            gather_dma(w + 1, 1, g1).start()
            gather_dma(w + 1, 1, g1).wait()
            write_dma(w + 1, 1, s1).start()

            @pl.when(w + 2 < nwin)
            def _():
                write_dma(w, 0, s0).wait()
                gather_dma(w + 2, 0, g0).start()

        write_dma(nwin - 2, 0, s0).wait()
        write_dma(nwin - 1, 1, s1).wait()

    return gather_kernel(table, idx_padded)


def _ce_body(emb_ref, tgt_ref, loss_ref):
    step = pl.program_id(0)

    @pl.when(step == 0)
    def _():
        loss_ref[0, 0] = 0.0

    rows = emb_ref[...]
    r, v = rows.shape
    m = jnp.max(rows, axis=1, keepdims=True)
    s = jnp.sum(jnp.exp(rows - m), axis=1)
    lse = m.reshape(r) + jnp.log(s)
    tg = tgt_ref[...]
    col_iota = jax.lax.broadcasted_iota(jnp.int32, (r, v), 1)
    tval = jnp.sum(jnp.where(col_iota == tg, rows, 0.0), axis=1)
    loss_ref[0, 0] += jnp.sum(lse - tval)


def _tc_ce(n, v, emb, tgtcol):
    loss_sum = pl.pallas_call(
        _ce_body,
        grid=(n // R,),
        in_specs=[
            pl.BlockSpec((R, v), lambda i: (i, 0)),
            pl.BlockSpec((R, 1), lambda i: (i, 0)),
        ],
        out_specs=pl.BlockSpec((1, 1), lambda i: (0, 0), memory_space=pltpu.SMEM),
        out_shape=jax.ShapeDtypeStruct((1, 1), jnp.float32),
    )(emb, tgtcol)
    return loss_sum[0, 0]


def kernel(indices, targets, table):
    b, t = indices.shape
    n = b * t
    vocab, v = table.shape
    sub = v // LANES
    tgtcol = targets.reshape(n, 1)
    idx_win4 = indices.reshape(n // GW, GW)
    idx_padded = jnp.pad(idx_win4, ((0, 0), (0, 8 - GW))).reshape(1, n // GW * 8)

    emb = _sc_gather(n, v, table, idx_padded)
    loss_sum = _tc_ce(n, v, emb, tgtcol)
    embeddings = emb.reshape(b, t, v)
    loss = loss_sum / n
    return (embeddings, loss)


# speedup vs baseline: 2.8514x; 1.0744x over previous
import jax
import jax.numpy as jnp
from jax.experimental import pallas as pl
from jax.experimental.pallas import tpu as pltpu
from jax.experimental.pallas import tpu_sc as plsc

LANES = 128
GW = 4
R = 512


def _sc_gather(n, v, table, idx_padded):
    n_units = 32
    per_unit = n // n_units
    stride = 8

    @pl.kernel(
        out_type=jax.ShapeDtypeStruct((n, v), jnp.float32),
        mesh=plsc.VectorSubcoreMesh(core_axis_name="c", subcore_axis_name="s"),
        scratch_types=[
            pltpu.VMEM((per_unit // GW * stride,), jnp.int32),
            pltpu.VMEM((2, GW, v), jnp.float32),
            pltpu.SemaphoreType.DMA,
            pltpu.SemaphoreType.DMA,
            pltpu.SemaphoreType.DMA,
            pltpu.SemaphoreType.DMA,
        ],
    )
    def gather_kernel(table_hbm, i_hbm, o_hbm, idx_vmem, buf, g0, g1, s0, s1):
        core = jax.lax.axis_index("c")
        subcore = jax.lax.axis_index("s")
        u = core * 16 + subcore
        base = u * per_unit
        nwin = per_unit // GW

        pltpu.async_copy(
            i_hbm.at[0, pl.ds(u * (nwin * stride), nwin * stride)], idx_vmem, g0
        ).wait()

        def gather_dma(w, slot, sem):
            idx_win = idx_vmem.at[pl.ds(w * stride, GW)]
            return pltpu.make_async_copy(table_hbm.at[idx_win], buf.at[slot], sem)

        def write_dma(w, slot, sem):
            return pltpu.make_async_copy(
                buf.at[slot], o_hbm.at[pl.ds(base + w * GW, GW)], sem
            )

        gather_dma(0, 0, g0).start()

        @pl.loop(0, nwin, step=2)
        def _(w):
            gather_dma(w, 0, g0).wait()
            write_dma(w, 0, s0).start()

            @pl.when(w > 0)
            def _():
                write_dma(w - 1, 1, s1).wait()

            gather_dma(w + 1, 1, g1).start()
            gather_dma(w + 1, 1, g1).wait()
            write_dma(w + 1, 1, s1).start()

            @pl.when(w + 2 < nwin)
            def _():
                write_dma(w, 0, s0).wait()
                gather_dma(w + 2, 0, g0).start()

        write_dma(nwin - 2, 0, s0).wait()
        write_dma(nwin - 1, 1, s1).wait()

    return gather_kernel(table, idx_padded)


def _ce_body(emb_ref, tgt_ref, loss_ref):
    step = pl.program_id(0)

    @pl.when(step == 0)
    def _():
        loss_ref[0, 0] = 0.0

    rows = emb_ref[...]
    r, v = rows.shape
    m = jnp.max(rows, axis=1, keepdims=True)
    s = jnp.sum(jnp.exp(rows - m), axis=1)
    lse = m.reshape(r) + jnp.log(s)
    tg = tgt_ref[...]
    col_iota = jax.lax.broadcasted_iota(jnp.int32, (r, v), 1)
    tval = jnp.sum(jnp.where(col_iota == tg, rows, 0.0), axis=1)
    loss_ref[0, 0] += jnp.sum(lse - tval)


def _tc_ce(n, v, emb, tgtcol):
    loss_sum = pl.pallas_call(
        _ce_body,
        grid=(n // R,),
        in_specs=[
            pl.BlockSpec((R, v), lambda i: (i, 0)),
            pl.BlockSpec((R, 1), lambda i: (i, 0)),
        ],
        out_specs=pl.BlockSpec((1, 1), lambda i: (0, 0), memory_space=pltpu.SMEM),
        out_shape=jax.ShapeDtypeStruct((1, 1), jnp.float32),
    )(emb, tgtcol)
    return loss_sum[0, 0]


def kernel(indices, targets, table):
    b, t = indices.shape
    n = b * t
    vocab, v = table.shape
    sub = v // LANES
    tgtcol = targets.reshape(n, 1)
    idx_win4 = indices.reshape(n // GW, GW)
    idx_padded = jnp.pad(idx_win4, ((0, 0), (0, 8 - GW))).reshape(1, n // GW * 8)

    emb = _sc_gather(n, v, table, idx_padded)
    loss_sum = _tc_ce(n, v, emb, tgtcol)
    embeddings = emb.reshape(b, t, v)
    loss = loss_sum / n
    return (embeddings, loss)
